# baseline (speedup 1.0000x reference)
import functools

import jax
import jax.numpy as jnp
from jax import lax
from jax.experimental import pallas as pl
from jax.experimental.pallas import tpu as pltpu

V = 1000
D = 128
B = 4096

_CB = 1024


def _table_body(e_ref, ctx_ref, out_ref, s_ref):
    @pl.when(pl.program_id(0) == 0)
    def _():
        s = lax.dot_general(
            e_ref[...], e_ref[...],
            (((1,), (1,)), ((), ())),
            preferred_element_type=jnp.float32,
        )
        ls = jnp.minimum(s, 0.0) - jnp.log1p(jnp.exp(-jnp.abs(s)))
        s_ref[...] = ls.astype(jnp.bfloat16)

    ctx = ctx_ref[0, :]
    onehot = (lax.broadcasted_iota(jnp.int32, (V, _CB), 0)
              == ctx[None, :]).astype(jnp.bfloat16)
    out_ref[...] = lax.dot_general(
        s_ref[...], onehot,
        (((1,), (0,)), ((), ())),
        preferred_element_type=jnp.float32,
    ).astype(jnp.bfloat16)


def _table(e, ctx_row):
    return pl.pallas_call(
        _table_body,
        grid=(B // _CB,),
        in_specs=[
            pl.BlockSpec((V, D), lambda j: (0, 0)),
            pl.BlockSpec((1, _CB), lambda j: (0, j)),
        ],
        out_specs=pl.BlockSpec((V, _CB), lambda j: (0, j)),
        out_shape=jax.ShapeDtypeStruct((V, B), jnp.bfloat16),
        scratch_shapes=[pltpu.VMEM((V, V), jnp.bfloat16)],
    )(e, ctx_row)


_RB = 256
_XB = 1024


def _expand_body(cen_ref, tab_ref, out_ref):
    cen = cen_ref[0, :]
    onehot = (cen[:, None]
              == lax.broadcasted_iota(jnp.int32, (_RB, V), 1)
              ).astype(jnp.bfloat16)
    out_ref[...] = lax.dot_general(
        onehot, tab_ref[...],
        (((1,), (0,)), ((), ())),
        preferred_element_type=jnp.float32,
    )


def _expand(cen_row, tab):
    return pl.pallas_call(
        _expand_body,
        grid=(B // _XB, B // _RB),
        in_specs=[
            pl.BlockSpec((1, _RB), lambda j, i: (0, i)),
            pl.BlockSpec((V, _XB), lambda j, i: (0, j)),
        ],
        out_specs=pl.BlockSpec((_RB, _XB), lambda j, i: (i, j)),
        out_shape=jax.ShapeDtypeStruct((B, B), jnp.float32),
    )(cen_row, tab)


def kernel(center_id, context_id, emb_table):
    tab = _table(emb_table, context_id.reshape(1, B))
    return _expand(center_id.reshape(1, B), tab)


# speedup vs baseline: 1.3727x; 1.3727x over previous
import functools

import jax
import jax.numpy as jnp
from jax import lax
from jax.experimental import pallas as pl
from jax.experimental.pallas import tpu as pltpu

V = 1000
D = 128
B = 4096

_XB = 1024
_RB = 512


def _fused_body(e_ref, ctx_ref, cen_ref, out_ref, s_ref, tab_ref):
    j = pl.program_id(0)
    i = pl.program_id(1)

    @pl.when(jnp.logical_and(j == 0, i == 0))
    def _():
        s = lax.dot_general(
            e_ref[...], e_ref[...],
            (((1,), (1,)), ((), ())),
            preferred_element_type=jnp.float32,
        )
        ls = jnp.minimum(s, 0.0) - jnp.log1p(jnp.exp(-jnp.abs(s)))
        s_ref[...] = ls.astype(jnp.bfloat16)

    @pl.when(i == 0)
    def _():
        ctx = ctx_ref[0, :]
        onehot_x = (lax.broadcasted_iota(jnp.int32, (V, _XB), 0)
                    == ctx[None, :]).astype(jnp.bfloat16)
        tab_ref[...] = lax.dot_general(
            s_ref[...], onehot_x,
            (((1,), (0,)), ((), ())),
            preferred_element_type=jnp.float32,
        ).astype(jnp.bfloat16)

    cen = cen_ref[0, :]
    onehot_c = (cen[:, None]
                == lax.broadcasted_iota(jnp.int32, (_RB, V), 1)
                ).astype(jnp.bfloat16)
    out_ref[...] = lax.dot_general(
        onehot_c, tab_ref[...],
        (((1,), (0,)), ((), ())),
        preferred_element_type=jnp.float32,
    )


def kernel(center_id, context_id, emb_table):
    return pl.pallas_call(
        _fused_body,
        grid=(B // _XB, B // _RB),
        in_specs=[
            pl.BlockSpec((V, D), lambda j, i: (0, 0)),
            pl.BlockSpec((1, _XB), lambda j, i: (0, j)),
            pl.BlockSpec((1, _RB), lambda j, i: (0, i)),
        ],
        out_specs=pl.BlockSpec((_RB, _XB), lambda j, i: (i, j)),
        out_shape=jax.ShapeDtypeStruct((B, B), jnp.float32),
        scratch_shapes=[
            pltpu.VMEM((V, V), jnp.bfloat16),
            pltpu.VMEM((V, _XB), jnp.bfloat16),
        ],
    )(emb_table, context_id.reshape(1, B), center_id.reshape(1, B))


# speedup vs baseline: 1.5010x; 1.0935x over previous
import functools

import jax
import jax.numpy as jnp
from jax import lax
from jax.experimental import pallas as pl
from jax.experimental.pallas import tpu as pltpu

V = 1000
D = 128
B = 4096

_RB = 256


def _fused_body(e_ref, ctx_ref, cen_ref, out_ref, s_ref, tab_ref):
    i = pl.program_id(0)

    @pl.when(i == 0)
    def _():
        s = lax.dot_general(
            e_ref[...], e_ref[...],
            (((1,), (1,)), ((), ())),
            preferred_element_type=jnp.float32,
        )
        ls = jnp.minimum(s, 0.0) - jnp.log1p(jnp.exp(-jnp.abs(s)))
        s_ref[...] = ls.astype(jnp.bfloat16)
        ctx = ctx_ref[0, :]
        onehot_x = (lax.broadcasted_iota(jnp.int32, (V, B), 0)
                    == ctx[None, :]).astype(jnp.bfloat16)
        tab_ref[...] = lax.dot_general(
            s_ref[...], onehot_x,
            (((1,), (0,)), ((), ())),
            preferred_element_type=jnp.float32,
        ).astype(jnp.bfloat16)

    cen = cen_ref[0, :]
    onehot_c = (cen[:, None]
                == lax.broadcasted_iota(jnp.int32, (_RB, V), 1)
                ).astype(jnp.bfloat16)
    out_ref[...] = lax.dot_general(
        onehot_c, tab_ref[...],
        (((1,), (0,)), ((), ())),
        preferred_element_type=jnp.float32,
    )


def kernel(center_id, context_id, emb_table):
    return pl.pallas_call(
        _fused_body,
        grid=(B // _RB,),
        in_specs=[
            pl.BlockSpec((V, D), lambda i: (0, 0)),
            pl.BlockSpec((1, B), lambda i: (0, 0)),
            pl.BlockSpec((1, _RB), lambda i: (0, i)),
        ],
        out_specs=pl.BlockSpec((_RB, B), lambda i: (i, 0)),
        out_shape=jax.ShapeDtypeStruct((B, B), jnp.float32),
        scratch_shapes=[
            pltpu.VMEM((V, V), jnp.bfloat16),
            pltpu.VMEM((V, B), jnp.bfloat16),
        ],
    )(emb_table, context_id.reshape(1, B), center_id.reshape(1, B))


# speedup vs baseline: 1.5134x; 1.0083x over previous
import functools

import jax
import jax.numpy as jnp
from jax import lax
from jax.experimental import pallas as pl
from jax.experimental.pallas import tpu as pltpu

V = 1000
D = 128
B = 4096

_RB = 512


def _fused_body(e_ref, ctx_ref, cen_ref, out_ref, s_ref, tab_ref):
    i = pl.program_id(0)

    @pl.when(i == 0)
    def _():
        s = lax.dot_general(
            e_ref[...], e_ref[...],
            (((1,), (1,)), ((), ())),
            preferred_element_type=jnp.float32,
        )
        ls = jnp.minimum(s, 0.0) - jnp.log1p(jnp.exp(-jnp.abs(s)))
        s_ref[...] = ls.astype(jnp.bfloat16)
        ctx = ctx_ref[0, :]
        onehot_x = (lax.broadcasted_iota(jnp.int32, (V, B), 0)
                    == ctx[None, :]).astype(jnp.bfloat16)
        tab_ref[...] = lax.dot_general(
            s_ref[...], onehot_x,
            (((1,), (0,)), ((), ())),
            preferred_element_type=jnp.float32,
        ).astype(jnp.bfloat16)

    cen = cen_ref[0, :]
    onehot_c = (cen[:, None]
                == lax.broadcasted_iota(jnp.int32, (_RB, V), 1)
                ).astype(jnp.bfloat16)
    out_ref[...] = lax.dot_general(
        onehot_c, tab_ref[...],
        (((1,), (0,)), ((), ())),
        preferred_element_type=jnp.float32,
    )


def kernel(center_id, context_id, emb_table):
    return pl.pallas_call(
        _fused_body,
        grid=(B // _RB,),
        in_specs=[
            pl.BlockSpec((V, D), lambda i: (0, 0)),
            pl.BlockSpec((1, B), lambda i: (0, 0)),
            pl.BlockSpec((1, _RB), lambda i: (0, i)),
        ],
        out_specs=pl.BlockSpec((_RB, B), lambda i: (i, 0)),
        out_shape=jax.ShapeDtypeStruct((B, B), jnp.float32),
        scratch_shapes=[
            pltpu.VMEM((V, V), jnp.bfloat16),
            pltpu.VMEM((V, B), jnp.bfloat16),
        ],
    )(emb_table, context_id.reshape(1, B), center_id.reshape(1, B))


# speedup vs baseline: 1.5194x; 1.0039x over previous
import jax
import jax.numpy as jnp
from jax import lax
from jax.experimental import pallas as pl
from jax.experimental.pallas import tpu as pltpu

V = 1000
D = 128
B = 4096

_RB = 512


def _fused_body(e_ref, ctx_ref, cen_ref, out_ref, s_ref, tab_ref):
    i = pl.program_id(0)

    @pl.when(i == 0)
    def _():
        s = lax.dot_general(
            e_ref[...], e_ref[...],
            (((1,), (1,)), ((), ())),
            preferred_element_type=jnp.float32,
        )
        ls = jnp.minimum(s, 0.0) - jnp.log1p(jnp.exp(-jnp.abs(s)))
        s_ref[...] = ls.astype(jnp.bfloat16)
        ctx = ctx_ref[0, :]
        onehot_x = (lax.broadcasted_iota(jnp.int32, (V, B), 0)
                    == ctx[None, :]).astype(jnp.bfloat16)
        tab_ref[...] = lax.dot_general(
            s_ref[...], onehot_x,
            (((1,), (0,)), ((), ())),
            preferred_element_type=jnp.float32,
        ).astype(jnp.bfloat16)

    cen = cen_ref[0, :]
    onehot_c = (cen[:, None]
                == lax.broadcasted_iota(jnp.int32, (_RB, V), 1)
                ).astype(jnp.bfloat16)
    out_ref[...] = lax.dot_general(
        onehot_c, tab_ref[...],
        (((1,), (0,)), ((), ())),
        preferred_element_type=jnp.float32,
    )


def kernel(center_id, context_id, emb_table):
    return pl.pallas_call(
        _fused_body,
        grid=(B // _RB,),
        in_specs=[
            pl.BlockSpec((V, D), lambda i: (0, 0)),
            pl.BlockSpec((1, B), lambda i: (0, 0)),
            pl.BlockSpec((1, _RB), lambda i: (0, i)),
        ],
        out_specs=pl.BlockSpec((_RB, B), lambda i: (i, 0)),
        out_shape=jax.ShapeDtypeStruct((B, B), jnp.float32),
        scratch_shapes=[
            pltpu.VMEM((V, V), jnp.bfloat16),
            pltpu.VMEM((V, B), jnp.bfloat16),
        ],
    )(emb_table, context_id.reshape(1, B), center_id.reshape(1, B))
